# baseline (speedup 1.0000x reference)
import functools

import jax
import jax.numpy as jnp
from jax import lax
from jax.experimental import pallas as pl
from jax.experimental.pallas import tpu as pltpu
from jax.experimental.pallas import tpu_sc as plsc

VOCAB = 1000000
EMBED = 64
BATCH = 16384

NC = 2
NS = 16
NW = NC * NS
BPW = BATCH // NW
CHUNK = 128
NCHUNK = BPW // CHUNK


@functools.cache
def _gather3():
  mesh = plsc.VectorSubcoreMesh(core_axis_name="c", subcore_axis_name="s")
  out = jax.ShapeDtypeStruct((BATCH, EMBED), jnp.float32)

  @functools.partial(
      pl.kernel,
      out_type=(out, out, out),
      mesh=mesh,
      compiler_params=pltpu.CompilerParams(use_tc_tiling_on_sc=False),
      scratch_types=[
          pltpu.VMEM((3 * NCHUNK, CHUNK), jnp.int32),
          pltpu.VMEM((3, BPW, EMBED), jnp.float32),
          pltpu.SemaphoreType.DMA,
          pltpu.SemaphoreType.DMA,
      ],
  )
  def body(in_tok, ctx_tok, neg_tok, w_in, w_ctx,
           out_in, out_ctx, out_neg, idx_v, rows_v, gsem, wsem):
    wid = lax.axis_index("s") * NC + lax.axis_index("c")
    base = wid * BPW
    toks = (in_tok, ctx_tok, neg_tok)
    tabs = (w_in, w_ctx, w_ctx)
    outs = (out_in, out_ctx, out_neg)

    for g in range(3):
      for c in range(NCHUNK):
        pltpu.sync_copy(toks[g].at[pl.ds(base + c * CHUNK, CHUNK)],
                        idx_v.at[g * NCHUNK + c])

    gathers = []
    for g in range(3):
      for c in range(NCHUNK):
        gathers.append(pltpu.async_copy(
            tabs[g].at[idx_v.at[g * NCHUNK + c]],
            rows_v.at[g, pl.ds(c * CHUNK, CHUNK)],
            gsem))

    writes = []
    for g in range(3):
      for c in range(NCHUNK):
        gathers[g * NCHUNK + c].wait()
      writes.append(pltpu.async_copy(
          rows_v.at[g], outs[g].at[pl.ds(base, BPW)], wsem))
    for w in writes:
      w.wait()

  return body


def kernel(input_tokens, context_tokens, negative_context, W_in, W_ctx):
  f = _gather3()
  return f(input_tokens.astype(jnp.int32),
           context_tokens.astype(jnp.int32),
           negative_context.astype(jnp.int32),
           W_in, W_ctx)
